# baseline (speedup 1.0000x reference)
import functools

import jax
import jax.numpy as jnp
from jax import lax
from jax.experimental import pallas as pl
from jax.experimental.pallas import tpu as pltpu
from jax.experimental.pallas import tpu_sc as plsc

_EPS = 1e-3
_H = 768
_NLANE = 16
_NJ = _H // _NLANE
_NW = 32
_B = 4
_S = 2048
_PPW = _S // _NW
_CHUNK = 16
_NCHUNK = _PPW // _CHUNK


def _shuf(v, idx):
    dnums = lax.GatherDimensionNumbers(
        offset_dims=(), collapsed_slice_dims=(0,), start_index_map=(0,))
    return lax.gather(v, idx[:, None], dnums, (1,),
                      mode=lax.GatherScatterMode.PROMISE_IN_BOUNDS)


def _lane_sum(v):
    lanes = lax.iota(jnp.int32, _NLANE)
    for k in (1, 2, 4, 8):
        v = v + _shuf(v, lanes ^ k)
    return v


def _nr_rsqrt(v):
    i = lax.bitcast_convert_type(v, jnp.int32)
    i = jnp.int32(0x5F3759DF) - lax.shift_right_arithmetic(i, 1)
    y = lax.bitcast_convert_type(i, jnp.float32)
    for _ in range(3):
        y = y * (1.5 - 0.5 * v * y * y)
    return y


def _sc_body(x_hbm, pos_hbm, g_hbm, b_hbm, out_hbm, pos_v, h_v, g_v, b_v):
    wid = lax.axis_index("s") * 2 + lax.axis_index("c")
    pltpu.sync_copy(g_hbm, g_v)
    pltpu.sync_copy(b_hbm, b_v)
    pbase = wid * _PPW

    def chunk_body(c, carry):
        pstart = pbase + c * _CHUNK
        pltpu.sync_copy(pos_hbm.at[pl.ds(pstart, _CHUNK)], pos_v)

        def batch_body(b, carry2):
            rstart = b * _S + pstart
            pltpu.sync_copy(x_hbm.at[pl.ds(rstart, _CHUNK)], h_v)

            def row_body(r, carry3):
                s = jnp.zeros((_NLANE,), jnp.float32)
                ss = jnp.zeros((_NLANE,), jnp.float32)
                for j in range(_NJ):
                    sl = pl.ds(j * _NLANE, _NLANE)
                    v = h_v[r, sl] + pos_v[r, sl]
                    h_v[r, sl] = v
                    s = s + v
                    ss = ss + v * v
                mean = _lane_sum(s) * (1.0 / _H)
                var = _lane_sum(ss) * (1.0 / _H) - mean * mean
                rinv = _nr_rsqrt(var + _EPS)
                a1 = rinv
                a0 = -mean * rinv
                for j in range(_NJ):
                    sl = pl.ds(j * _NLANE, _NLANE)
                    h_v[r, sl] = (h_v[r, sl] * a1 + a0) * g_v[sl] + b_v[sl]
                return carry3

            lax.fori_loop(0, _CHUNK, row_body, 0)
            pltpu.sync_copy(h_v, out_hbm.at[pl.ds(rstart, _CHUNK)])
            return carry2

        lax.fori_loop(0, _B, batch_body, 0)
        return carry

    lax.fori_loop(0, _NCHUNK, chunk_body, 0)


def kernel(x, pos_table, gamma, beta):
    B, S, H = x.shape
    x2 = x.reshape(B * S, H)
    k = pl.kernel(
        _sc_body,
        out_type=jax.ShapeDtypeStruct((B * S, H), jnp.float32),
        mesh=plsc.VectorSubcoreMesh(core_axis_name="c", subcore_axis_name="s"),
        scratch_types=[
            pltpu.VMEM((_CHUNK, H), jnp.float32),
            pltpu.VMEM((_CHUNK, H), jnp.float32),
            pltpu.VMEM((H,), jnp.float32),
            pltpu.VMEM((H,), jnp.float32),
        ],
    )
    out = k(x2, pos_table, gamma, beta)
    return out.reshape(B, S, H)
